# baseline (speedup 1.0000x reference)
import functools

import jax
import jax.numpy as jnp
from jax import lax
from jax.experimental import pallas as pl
from jax.experimental.pallas import tpu as pltpu
from jax.experimental.pallas import tpu_sc as plsc

B = 16384
EMB = 64
NUM = 16
HID = 256

_CH = 128


def _build_gather():
    info = plsc.get_sparse_core_info()
    nc, ns = info.num_cores, info.num_subcores
    nw = nc * ns
    bw = B // nw
    n_ch = bw // _CH

    mesh = plsc.VectorSubcoreMesh(core_axis_name="c", subcore_axis_name="s")

    @functools.partial(
        pl.kernel,
        mesh=mesh,
        compiler_params=pltpu.CompilerParams(use_tc_tiling_on_sc=False),
        out_type=[jax.ShapeDtypeStruct((B, EMB), jnp.bfloat16)] * 4,
        scratch_types=[
            pltpu.VMEM((n_ch, _CH), jnp.int32),
            pltpu.VMEM((bw, EMB), jnp.bfloat16),
            pltpu.SemaphoreType.DMA,
        ],
    )
    def gather_kernel(t_item, t_cat, t_brand, t_shop,
                      i_item, i_cat, i_brand, i_shop,
                      o_item, o_cat, o_brand, o_shop,
                      idx_v, rows_v, sem):
        wid = lax.axis_index("s") * nc + lax.axis_index("c")
        base = wid * bw
        for tbl, idx_h, out_h in ((t_item, i_item, o_item),
                                  (t_cat, i_cat, o_cat),
                                  (t_brand, i_brand, o_brand),
                                  (t_shop, i_shop, o_shop)):
            pltpu.sync_copy(idx_h.at[pl.ds(wid * n_ch, n_ch)], idx_v)
            copies = []
            for j in range(n_ch):
                copies.append(pltpu.async_copy(
                    tbl.at[idx_v.at[j]],
                    rows_v.at[pl.ds(j * _CH, _CH)],
                    sem))
            for c in copies:
                c.wait()
            pltpu.sync_copy(rows_v, out_h.at[pl.ds(base, bw)])

    return gather_kernel


_gather = _build_gather()


def _mm_body(numt_ref, e1_ref, e2_ref, e3_ref, e4_ref,
             wn_ref, w1_ref, w2_ref, w3_ref, w4_ref, b_ref, out_ref):
    acc = lax.dot_general(numt_ref[...], wn_ref[...], (((0,), (0,)), ((), ())),
                          preferred_element_type=jnp.float32)
    acc += jnp.dot(e1_ref[...].astype(jnp.float32), w1_ref[...],
                   preferred_element_type=jnp.float32)
    acc += jnp.dot(e2_ref[...].astype(jnp.float32), w2_ref[...],
                   preferred_element_type=jnp.float32)
    acc += jnp.dot(e3_ref[...].astype(jnp.float32), w3_ref[...],
                   preferred_element_type=jnp.float32)
    acc += jnp.dot(e4_ref[...].astype(jnp.float32), w4_ref[...],
                   preferred_element_type=jnp.float32)
    out_ref[...] = acc + b_ref[...]


_BM = 2048


def _dense(num_t, e1, e2, e3, e4, wn, w1, w2, w3, w4, b2):
    grid = (B // _BM,)
    row_spec = pl.BlockSpec((_BM, EMB), lambda i: (i, 0))
    full = lambda s: pl.BlockSpec(s, lambda i: (0, 0))
    return pl.pallas_call(
        _mm_body,
        grid=grid,
        in_specs=[
            pl.BlockSpec((NUM, _BM), lambda i: (0, i)),
            row_spec, row_spec, row_spec, row_spec,
            full((NUM, HID)), full((EMB, HID)), full((EMB, HID)),
            full((EMB, HID)), full((EMB, HID)), full((1, HID)),
        ],
        out_specs=pl.BlockSpec((_BM, HID), lambda i: (i, 0)),
        out_shape=jax.ShapeDtypeStruct((B, HID), jnp.float32),
    )(num_t, e1, e2, e3, e4, wn, w1, w2, w3, w4, b2)


def kernel(numerical, idx_item_id, idx_category_id, idx_brand_id,
           idx_shop_id, emb_item_id, emb_category_id, emb_brand_id,
           emb_shop_id, W, b):
    to_idx = lambda i: i.astype(jnp.int32).reshape(-1, _CH)
    e1, e2, e3, e4 = _gather(
        emb_item_id.astype(jnp.bfloat16), emb_category_id.astype(jnp.bfloat16),
        emb_brand_id.astype(jnp.bfloat16), emb_shop_id.astype(jnp.bfloat16),
        to_idx(idx_item_id), to_idx(idx_category_id),
        to_idx(idx_brand_id), to_idx(idx_shop_id))
    wt = W.T
    wn = wt[:NUM]
    w1 = wt[NUM:NUM + EMB]
    w2 = wt[NUM + EMB:NUM + 2 * EMB]
    w3 = wt[NUM + 2 * EMB:NUM + 3 * EMB]
    w4 = wt[NUM + 3 * EMB:]
    return _dense(numerical.T, e1, e2, e3, e4, wn, w1, w2, w3, w4,
                  b.reshape(1, HID))


# speedup vs baseline: 1.5419x; 1.5419x over previous
import functools

import jax
import jax.numpy as jnp
from jax import lax
from jax.experimental import pallas as pl
from jax.experimental.pallas import tpu as pltpu
from jax.experimental.pallas import tpu_sc as plsc

B = 16384
EMB = 64
NUM = 16
HID = 256
W128 = 128

_CH = 128


def _build_gather():
    info = plsc.get_sparse_core_info()
    nc, ns = info.num_cores, info.num_subcores
    nw = nc * ns
    bw = B // nw
    n_ch = bw // _CH

    mesh = plsc.VectorSubcoreMesh(core_axis_name="c", subcore_axis_name="s")

    @functools.partial(
        pl.kernel,
        mesh=mesh,
        compiler_params=pltpu.CompilerParams(use_tc_tiling_on_sc=False),
        out_type=[jax.ShapeDtypeStruct((B, W128), jnp.float32)] * 4,
        scratch_types=[
            pltpu.VMEM((n_ch, _CH), jnp.int32),
            pltpu.VMEM((bw, W128), jnp.float32),
            pltpu.SemaphoreType.DMA,
        ],
    )
    def gather_kernel(t_item, t_cat, t_brand, t_shop,
                      i_item, i_cat, i_brand, i_shop,
                      o_item, o_cat, o_brand, o_shop,
                      idx_v, rows_v, sem):
        wid = lax.axis_index("s") * nc + lax.axis_index("c")
        base = wid * bw
        for tbl, idx_h, out_h in ((t_item, i_item, o_item),
                                  (t_cat, i_cat, o_cat),
                                  (t_brand, i_brand, o_brand),
                                  (t_shop, i_shop, o_shop)):
            pltpu.sync_copy(idx_h.at[pl.ds(wid * n_ch, n_ch)], idx_v)
            copies = []
            for j in range(n_ch):
                copies.append(pltpu.async_copy(
                    tbl.at[idx_v.at[j]],
                    rows_v.at[pl.ds(j * _CH, _CH)],
                    sem))
            for c in copies:
                c.wait()
            pltpu.sync_copy(rows_v, out_h.at[pl.ds(base, bw)])

    return gather_kernel


_gather = _build_gather()


def _prep_body(embt_ref, out_ref):
    t = embt_ref[...]
    bn = t.shape[1]
    out_ref[...] = jnp.concatenate(
        [t.T, jnp.zeros((bn, W128 - EMB), jnp.float32)], axis=1)


def _prep(emb_t, vocab, bn):
    grid = (pl.cdiv(vocab, bn),)
    return pl.pallas_call(
        _prep_body,
        grid=grid,
        in_specs=[pl.BlockSpec((EMB, bn), lambda i: (0, i))],
        out_specs=pl.BlockSpec((bn, W128), lambda i: (i, 0)),
        out_shape=jax.ShapeDtypeStruct((vocab, W128), jnp.float32),
    )(emb_t)


def _mm_body(numt_ref, e1_ref, e2_ref, e3_ref, e4_ref,
             wn_ref, w1_ref, w2_ref, w3_ref, w4_ref, b_ref, out_ref):
    acc = lax.dot_general(numt_ref[...], wn_ref[...], (((0,), (0,)), ((), ())),
                          preferred_element_type=jnp.float32)
    acc += jnp.dot(e1_ref[...], w1_ref[...], preferred_element_type=jnp.float32)
    acc += jnp.dot(e2_ref[...], w2_ref[...], preferred_element_type=jnp.float32)
    acc += jnp.dot(e3_ref[...], w3_ref[...], preferred_element_type=jnp.float32)
    acc += jnp.dot(e4_ref[...], w4_ref[...], preferred_element_type=jnp.float32)
    out_ref[...] = acc + b_ref[...]


_BM = 2048


def _dense(num_t, e1, e2, e3, e4, wn, w1, w2, w3, w4, b2):
    grid = (B // _BM,)
    row_spec = pl.BlockSpec((_BM, W128), lambda i: (i, 0))
    full = lambda s: pl.BlockSpec(s, lambda i: (0, 0))
    return pl.pallas_call(
        _mm_body,
        grid=grid,
        in_specs=[
            pl.BlockSpec((NUM, _BM), lambda i: (0, i)),
            row_spec, row_spec, row_spec, row_spec,
            full((NUM, HID)), full((W128, HID)), full((W128, HID)),
            full((W128, HID)), full((W128, HID)), full((1, HID)),
        ],
        out_specs=pl.BlockSpec((_BM, HID), lambda i: (i, 0)),
        out_shape=jax.ShapeDtypeStruct((B, HID), jnp.float32),
    )(num_t, e1, e2, e3, e4, wn, w1, w2, w3, w4, b2)


def kernel(numerical, idx_item_id, idx_category_id, idx_brand_id,
           idx_shop_id, emb_item_id, emb_category_id, emb_brand_id,
           emb_shop_id, W, b):
    to_idx = lambda i: i.astype(jnp.int32).reshape(-1, _CH)
    e1, e2, e3, e4 = _gather(
        _prep(emb_item_id.T, 1000000, 2048),
        _prep(emb_category_id.T, 1000, 1000),
        _prep(emb_brand_id.T, 100000, 2048),
        _prep(emb_shop_id.T, 100000, 2048),
        to_idx(idx_item_id), to_idx(idx_category_id),
        to_idx(idx_brand_id), to_idx(idx_shop_id))
    wt = W.T
    wpad = lambda w: jnp.pad(w, ((0, W128 - EMB), (0, 0)))
    wn = wt[:NUM]
    w1 = wpad(wt[NUM:NUM + EMB])
    w2 = wpad(wt[NUM + EMB:NUM + 2 * EMB])
    w3 = wpad(wt[NUM + 2 * EMB:NUM + 3 * EMB])
    w4 = wpad(wt[NUM + 3 * EMB:])
    return _dense(numerical.T, e1, e2, e3, e4, wn, w1, w2, w3, w4,
                  b.reshape(1, HID))


# speedup vs baseline: 2.4752x; 1.6054x over previous
import functools

import jax
import jax.numpy as jnp
from jax import lax
from jax.experimental import pallas as pl
from jax.experimental.pallas import tpu as pltpu
from jax.experimental.pallas import tpu_sc as plsc

B = 16384
EMB = 64
NUM = 16
HID = 256
W128 = 128

_CH = 128


def _build_gather():
    info = plsc.get_sparse_core_info()
    nc, ns = info.num_cores, info.num_subcores
    nw = nc * ns
    bw = B // nw
    n_ch = bw // _CH

    mesh = plsc.VectorSubcoreMesh(core_axis_name="c", subcore_axis_name="s")

    @functools.partial(
        pl.kernel,
        mesh=mesh,
        compiler_params=pltpu.CompilerParams(use_tc_tiling_on_sc=False),
        out_type=[jax.ShapeDtypeStruct((B, W128), jnp.float32)] * 4,
        scratch_types=[
            pltpu.VMEM((n_ch, _CH), jnp.int32),
            pltpu.VMEM((bw, W128), jnp.float32),
            pltpu.SemaphoreType.DMA,
        ],
    )
    def gather_kernel(t_item, t_cat, t_brand, t_shop,
                      i_item, i_cat, i_brand, i_shop,
                      o_item, o_cat, o_brand, o_shop,
                      idx_v, rows_v, sem):
        wid = lax.axis_index("s") * nc + lax.axis_index("c")
        base = wid * bw
        for tbl, idx_h, out_h in ((t_item, i_item, o_item),
                                  (t_cat, i_cat, o_cat),
                                  (t_brand, i_brand, o_brand),
                                  (t_shop, i_shop, o_shop)):
            pltpu.sync_copy(idx_h.at[pl.ds(wid * n_ch, n_ch)], idx_v)
            copies = []
            for j in range(n_ch):
                copies.append(pltpu.async_copy(
                    tbl.at[idx_v.at[j]],
                    rows_v.at[pl.ds(j * _CH, _CH)],
                    sem))
            for c in copies:
                c.wait()
            pltpu.sync_copy(rows_v, out_h.at[pl.ds(base, bw)])

    return gather_kernel


_gather = _build_gather()


def _prep_body(embt_ref, out_ref):
    t = embt_ref[...]
    bn = t.shape[1]
    out_ref[...] = jnp.concatenate(
        [t.T, jnp.zeros((bn, W128 - EMB), jnp.float32)], axis=1)


def _prep(emb_t, vocab, bn):
    grid = (pl.cdiv(vocab, bn),)
    return pl.pallas_call(
        _prep_body,
        grid=grid,
        in_specs=[pl.BlockSpec((EMB, bn), lambda i: (0, i))],
        out_specs=pl.BlockSpec((bn, W128), lambda i: (i, 0)),
        out_shape=jax.ShapeDtypeStruct((vocab, W128), jnp.float32),
    )(emb_t)


def _mm_body(numt_ref, e1_ref, e2_ref, e3_ref, e4_ref,
             wn_ref, w1_ref, w2_ref, w3_ref, w4_ref, b_ref, out_ref):
    acc = lax.dot_general(numt_ref[...], wn_ref[...], (((0,), (0,)), ((), ())),
                          preferred_element_type=jnp.float32)
    acc += jnp.dot(e1_ref[...], w1_ref[...], preferred_element_type=jnp.float32)
    acc += jnp.dot(e2_ref[...], w2_ref[...], preferred_element_type=jnp.float32)
    acc += jnp.dot(e3_ref[...], w3_ref[...], preferred_element_type=jnp.float32)
    acc += jnp.dot(e4_ref[...], w4_ref[...], preferred_element_type=jnp.float32)
    out_ref[...] = acc + b_ref[...]


_BM = 2048


def _dense(num_t, e1, e2, e3, e4, wn, w1, w2, w3, w4, b2):
    grid = (B // _BM,)
    row_spec = pl.BlockSpec((_BM, W128), lambda i: (i, 0))
    full = lambda s: pl.BlockSpec(s, lambda i: (0, 0))
    return pl.pallas_call(
        _mm_body,
        grid=grid,
        in_specs=[
            pl.BlockSpec((NUM, _BM), lambda i: (0, i)),
            row_spec, row_spec, row_spec, row_spec,
            full((NUM, HID)), full((W128, HID)), full((W128, HID)),
            full((W128, HID)), full((W128, HID)), full((1, HID)),
        ],
        out_specs=pl.BlockSpec((_BM, HID), lambda i: (i, 0)),
        out_shape=jax.ShapeDtypeStruct((B, HID), jnp.float32),
    )(num_t, e1, e2, e3, e4, wn, w1, w2, w3, w4, b2)


def kernel(numerical, idx_item_id, idx_category_id, idx_brand_id,
           idx_shop_id, emb_item_id, emb_category_id, emb_brand_id,
           emb_shop_id, W, b):
    to_idx = lambda i: i.astype(jnp.int32).reshape(-1, _CH)
    e1, e2, e3, e4 = _gather(
        _prep(emb_item_id.T, 1000000, 8192),
        _prep(emb_category_id.T, 1000, 1000),
        _prep(emb_brand_id.T, 100000, 8192),
        _prep(emb_shop_id.T, 100000, 8192),
        to_idx(idx_item_id), to_idx(idx_category_id),
        to_idx(idx_brand_id), to_idx(idx_shop_id))
    wt = W.T
    wpad = lambda w: jnp.pad(w, ((0, W128 - EMB), (0, 0)))
    wn = wt[:NUM]
    w1 = wpad(wt[NUM:NUM + EMB])
    w2 = wpad(wt[NUM + EMB:NUM + 2 * EMB])
    w3 = wpad(wt[NUM + 2 * EMB:NUM + 3 * EMB])
    w4 = wpad(wt[NUM + 3 * EMB:])
    return _dense(numerical.T, e1, e2, e3, e4, wn, w1, w2, w3, w4,
                  b.reshape(1, HID))


# speedup vs baseline: 2.6384x; 1.0659x over previous
import functools

import jax
import jax.numpy as jnp
from jax import lax
from jax.experimental import pallas as pl
from jax.experimental.pallas import tpu as pltpu
from jax.experimental.pallas import tpu_sc as plsc

B = 16384
EMB = 64
NUM = 16
HID = 256
W128 = 128

_CH = 128


def _build_gather():
    info = plsc.get_sparse_core_info()
    nc, ns = info.num_cores, info.num_subcores
    nw = nc * ns
    bw = B // nw
    n_ch = bw // _CH

    mesh = plsc.VectorSubcoreMesh(core_axis_name="c", subcore_axis_name="s")

    @functools.partial(
        pl.kernel,
        mesh=mesh,
        compiler_params=pltpu.CompilerParams(use_tc_tiling_on_sc=False),
        out_type=[jax.ShapeDtypeStruct((B, W128), jnp.float32)] * 4,
        scratch_types=[
            pltpu.VMEM((n_ch, _CH), jnp.int32),
            pltpu.VMEM((bw, W128), jnp.float32),
            pltpu.SemaphoreType.DMA,
        ],
    )
    def gather_kernel(t_item, t_cat, t_brand, t_shop,
                      i_item, i_cat, i_brand, i_shop,
                      o_item, o_cat, o_brand, o_shop,
                      idx_v, rows_v, sem):
        wid = lax.axis_index("s") * nc + lax.axis_index("c")
        base = wid * bw
        for tbl, idx_h, out_h in ((t_item, i_item, o_item),
                                  (t_cat, i_cat, o_cat),
                                  (t_brand, i_brand, o_brand),
                                  (t_shop, i_shop, o_shop)):
            pltpu.sync_copy(idx_h.at[pl.ds(wid * n_ch, n_ch)], idx_v)
            copies = []
            for j in range(n_ch):
                copies.append(pltpu.async_copy(
                    tbl.at[idx_v.at[j]],
                    rows_v.at[pl.ds(j * _CH, _CH)],
                    sem))
            for c in copies:
                c.wait()
            pltpu.sync_copy(rows_v, out_h.at[pl.ds(base, bw)])

    return gather_kernel


_gather = _build_gather()


def _prep_body(embt_ref, out_ref):
    t = embt_ref[...]
    bn = t.shape[1]
    out_ref[...] = jnp.concatenate(
        [t.T, jnp.zeros((bn, W128 - EMB), jnp.float32)], axis=1)


def _prep(emb_t, vocab, bn):
    grid = (pl.cdiv(vocab, bn),)
    return pl.pallas_call(
        _prep_body,
        grid=grid,
        in_specs=[pl.BlockSpec((EMB, bn), lambda i: (0, i))],
        out_specs=pl.BlockSpec((bn, W128), lambda i: (i, 0)),
        out_shape=jax.ShapeDtypeStruct((vocab, W128), jnp.float32),
    )(emb_t)


def _mm_body(numt_ref, e1_ref, e2_ref, e3_ref, e4_ref,
             wn_ref, w1_ref, w2_ref, w3_ref, w4_ref, b_ref, out_ref):
    acc = lax.dot_general(numt_ref[...], wn_ref[...], (((0,), (0,)), ((), ())),
                          preferred_element_type=jnp.float32)
    acc += jnp.dot(e1_ref[...], w1_ref[...], preferred_element_type=jnp.float32)
    acc += jnp.dot(e2_ref[...], w2_ref[...], preferred_element_type=jnp.float32)
    acc += jnp.dot(e3_ref[...], w3_ref[...], preferred_element_type=jnp.float32)
    acc += jnp.dot(e4_ref[...], w4_ref[...], preferred_element_type=jnp.float32)
    out_ref[...] = acc + b_ref[...]


_BM = 2048


def _dense(num_t, e1, e2, e3, e4, wn, w1, w2, w3, w4, b2):
    grid = (B // _BM,)
    row_spec = pl.BlockSpec((_BM, W128), lambda i: (i, 0))
    full = lambda s: pl.BlockSpec(s, lambda i: (0, 0))
    return pl.pallas_call(
        _mm_body,
        grid=grid,
        in_specs=[
            pl.BlockSpec((NUM, _BM), lambda i: (0, i)),
            row_spec, row_spec, row_spec, row_spec,
            full((NUM, HID)), full((W128, HID)), full((W128, HID)),
            full((W128, HID)), full((W128, HID)), full((1, HID)),
        ],
        out_specs=pl.BlockSpec((_BM, HID), lambda i: (i, 0)),
        out_shape=jax.ShapeDtypeStruct((B, HID), jnp.float32),
    )(num_t, e1, e2, e3, e4, wn, w1, w2, w3, w4, b2)


def kernel(numerical, idx_item_id, idx_category_id, idx_brand_id,
           idx_shop_id, emb_item_id, emb_category_id, emb_brand_id,
           emb_shop_id, W, b):
    to_idx = lambda i: i.astype(jnp.int32).reshape(-1, _CH)
    e1, e2, e3, e4 = _gather(
        _prep(emb_item_id.T, 1000000, 16384),
        _prep(emb_category_id.T, 1000, 1000),
        _prep(emb_brand_id.T, 100000, 16384),
        _prep(emb_shop_id.T, 100000, 16384),
        to_idx(idx_item_id), to_idx(idx_category_id),
        to_idx(idx_brand_id), to_idx(idx_shop_id))
    wt = W.T
    wpad = lambda w: jnp.pad(w, ((0, W128 - EMB), (0, 0)))
    wn = wt[:NUM]
    w1 = wpad(wt[NUM:NUM + EMB])
    w2 = wpad(wt[NUM + EMB:NUM + 2 * EMB])
    w3 = wpad(wt[NUM + 2 * EMB:NUM + 3 * EMB])
    w4 = wpad(wt[NUM + 3 * EMB:])
    return _dense(numerical.T, e1, e2, e3, e4, wn, w1, w2, w3, w4,
                  b.reshape(1, HID))


# speedup vs baseline: 2.6880x; 1.0188x over previous
import functools

import jax
import jax.numpy as jnp
from jax import lax
from jax.experimental import pallas as pl
from jax.experimental.pallas import tpu as pltpu
from jax.experimental.pallas import tpu_sc as plsc

B = 16384
EMB = 64
NUM = 16
HID = 256
W128 = 128

_CH = 128


def _build_gather():
    info = plsc.get_sparse_core_info()
    nc, ns = info.num_cores, info.num_subcores
    nw = nc * ns
    bw = B // nw
    n_ch = bw // _CH

    mesh = plsc.VectorSubcoreMesh(core_axis_name="c", subcore_axis_name="s")

    @functools.partial(
        pl.kernel,
        mesh=mesh,
        compiler_params=pltpu.CompilerParams(use_tc_tiling_on_sc=False),
        out_type=[jax.ShapeDtypeStruct((B, W128), jnp.float32)] * 4,
        scratch_types=[
            pltpu.VMEM((n_ch, _CH), jnp.int32),
            pltpu.VMEM((bw, W128), jnp.float32),
            pltpu.SemaphoreType.DMA,
        ],
    )
    def gather_kernel(t_item, t_cat, t_brand, t_shop,
                      i_item, i_cat, i_brand, i_shop,
                      o_item, o_cat, o_brand, o_shop,
                      idx_v, rows_v, sem):
        wid = lax.axis_index("s") * nc + lax.axis_index("c")
        base = wid * bw
        for tbl, idx_h, out_h in ((t_item, i_item, o_item),
                                  (t_cat, i_cat, o_cat),
                                  (t_brand, i_brand, o_brand),
                                  (t_shop, i_shop, o_shop)):
            pltpu.sync_copy(idx_h.at[pl.ds(wid * n_ch, n_ch)], idx_v)
            copies = []
            for j in range(n_ch):
                copies.append(pltpu.async_copy(
                    tbl.at[idx_v.at[j]],
                    rows_v.at[pl.ds(j * _CH, _CH)],
                    sem))
            for c in copies:
                c.wait()
            pltpu.sync_copy(rows_v, out_h.at[pl.ds(base, bw)])

    return gather_kernel


_gather = _build_gather()


def _prep_body(embt_ref, out_ref):
    t = embt_ref[...]
    bn = t.shape[1]
    out_ref[...] = jnp.concatenate(
        [t.T, jnp.zeros((bn, W128 - EMB), jnp.float32)], axis=1)


def _prep(emb_t, vocab, bn):
    grid = (pl.cdiv(vocab, bn),)
    return pl.pallas_call(
        _prep_body,
        grid=grid,
        in_specs=[pl.BlockSpec((EMB, bn), lambda i: (0, i))],
        out_specs=pl.BlockSpec((bn, W128), lambda i: (i, 0)),
        out_shape=jax.ShapeDtypeStruct((vocab, W128), jnp.float32),
    )(emb_t)


def _mm_body(numt_ref, e1_ref, e2_ref, e3_ref, e4_ref,
             wn_ref, w1_ref, w2_ref, w3_ref, w4_ref, b_ref, out_ref):
    acc = lax.dot_general(numt_ref[...], wn_ref[...], (((0,), (0,)), ((), ())),
                          preferred_element_type=jnp.float32)
    acc += jnp.dot(e1_ref[...], w1_ref[...], preferred_element_type=jnp.float32)
    acc += jnp.dot(e2_ref[...], w2_ref[...], preferred_element_type=jnp.float32)
    acc += jnp.dot(e3_ref[...], w3_ref[...], preferred_element_type=jnp.float32)
    acc += jnp.dot(e4_ref[...], w4_ref[...], preferred_element_type=jnp.float32)
    out_ref[...] = acc + b_ref[...]


_BM = 2048


def _dense(num_t, e1, e2, e3, e4, wn, w1, w2, w3, w4, b2):
    grid = (B // _BM,)
    row_spec = pl.BlockSpec((_BM, W128), lambda i: (i, 0))
    full = lambda s: pl.BlockSpec(s, lambda i: (0, 0))
    return pl.pallas_call(
        _mm_body,
        grid=grid,
        in_specs=[
            pl.BlockSpec((NUM, _BM), lambda i: (0, i)),
            row_spec, row_spec, row_spec, row_spec,
            full((NUM, HID)), full((W128, HID)), full((W128, HID)),
            full((W128, HID)), full((W128, HID)), full((1, HID)),
        ],
        out_specs=pl.BlockSpec((_BM, HID), lambda i: (i, 0)),
        out_shape=jax.ShapeDtypeStruct((B, HID), jnp.float32),
    )(num_t, e1, e2, e3, e4, wn, w1, w2, w3, w4, b2)


def kernel(numerical, idx_item_id, idx_category_id, idx_brand_id,
           idx_shop_id, emb_item_id, emb_category_id, emb_brand_id,
           emb_shop_id, W, b):
    to_idx = lambda i: i.astype(jnp.int32).reshape(-1, _CH)
    e1, e2, e3, e4 = _gather(
        _prep(emb_item_id.T, 1000000, 32768),
        _prep(emb_category_id.T, 1000, 1000),
        _prep(emb_brand_id.T, 100000, 32768),
        _prep(emb_shop_id.T, 100000, 32768),
        to_idx(idx_item_id), to_idx(idx_category_id),
        to_idx(idx_brand_id), to_idx(idx_shop_id))
    wt = W.T
    wpad = lambda w: jnp.pad(w, ((0, W128 - EMB), (0, 0)))
    wn = wt[:NUM]
    w1 = wpad(wt[NUM:NUM + EMB])
    w2 = wpad(wt[NUM + EMB:NUM + 2 * EMB])
    w3 = wpad(wt[NUM + 2 * EMB:NUM + 3 * EMB])
    w4 = wpad(wt[NUM + 3 * EMB:])
    return _dense(numerical.T, e1, e2, e3, e4, wn, w1, w2, w3, w4,
                  b.reshape(1, HID))


# speedup vs baseline: 2.8930x; 1.0763x over previous
import functools

import jax
import jax.numpy as jnp
from jax import lax
from jax.experimental import pallas as pl
from jax.experimental.pallas import tpu as pltpu
from jax.experimental.pallas import tpu_sc as plsc

B = 16384
EMB = 64
NUM = 16
HID = 256
W128 = 128

_BN = 32768
_HB = _BN // 2
_CH = 128
_HC = 256

V_ITEM = 1000000
V_CAT = 1000
V_BS = 100000
_NBI = -(-V_ITEM // 32768)
_SLOTS_ITEM = _NBI * (32768 // 2)


def _build_gather():
    info = plsc.get_sparse_core_info()
    nc, ns = info.num_cores, info.num_subcores
    nw = nc * ns
    bw = B // nw
    n_ch = bw // _CH
    n_sub = bw // _HC
    n_hch = _HC // _CH

    mesh = plsc.VectorSubcoreMesh(core_axis_name="c", subcore_axis_name="s")

    @functools.partial(
        pl.kernel,
        mesh=mesh,
        compiler_params=pltpu.CompilerParams(use_tc_tiling_on_sc=False),
        out_type=[jax.ShapeDtypeStruct((B, W128), jnp.float32)] * 3,
        scratch_types=[
            pltpu.VMEM((n_hch, _CH), jnp.int32),
            pltpu.VMEM((n_hch, _CH), jnp.int32),
            pltpu.VMEM((_HC, W128), jnp.float32),
            pltpu.VMEM((_HC, W128), jnp.float32),
            pltpu.SemaphoreType.DMA,
            pltpu.SemaphoreType.DMA,
        ],
    )
    def gather_kernel(t_item, t_cat, t_bs,
                      i_item, i_cat, i_brand, i_shop,
                      o_item, o_cat, o_bs,
                      idx_h, idx_h2, rows_a, rows_b, sem_a, sem_b):
        wid = lax.axis_index("s") * nc + lax.axis_index("c")
        base = wid * bw

        def fire(tbl, ih, crow, rows, sem, ibuf):
            pltpu.sync_copy(ih.at[pl.ds(crow, n_hch)], ibuf)
            copies = []
            for j in range(n_hch):
                copies.append(pltpu.async_copy(
                    tbl.at[ibuf.at[j]],
                    rows.at[pl.ds(j * _CH, _CH)], sem))
            return copies

        for tbl, ih, out_h in ((t_item, i_item, o_item),
                               (t_cat, i_cat, o_cat)):
            for s in range(n_sub):
                crow = wid * n_ch + s * n_hch
                for c in fire(tbl, ih, crow, rows_a, sem_a, idx_h):
                    c.wait()
                pltpu.sync_copy(rows_a,
                                out_h.at[pl.ds(base + s * _HC, _HC)])

        def pack(i, _):
            for k in range(EMB // 16):
                rows_a[i, pl.ds(EMB + k * 16, 16)] = (
                    rows_b[i, pl.ds(EMB + k * 16, 16)])
            return 0

        for s in range(n_sub):
            crow = wid * n_ch + s * n_hch
            cps = fire(t_bs, i_brand, crow, rows_a, sem_a, idx_h)
            cps += fire(t_bs, i_shop, crow, rows_b, sem_b, idx_h2)
            for c in cps:
                c.wait()
            lax.fori_loop(0, _HC, pack, 0)
            pltpu.sync_copy(rows_a, o_bs.at[pl.ds(base + s * _HC, _HC)])

    return gather_kernel


_gather = _build_gather()


def _prep_pair_body(t1_ref, t2_ref, out_ref):
    out_ref[...] = jnp.concatenate([t1_ref[...].T, t2_ref[...].T], axis=1)


def _prep_item(emb_t):
    grid = (_NBI,)
    return pl.pallas_call(
        _prep_pair_body,
        grid=grid,
        in_specs=[pl.BlockSpec((EMB, _HB), lambda i: (0, 2 * i)),
                  pl.BlockSpec((EMB, _HB), lambda i: (0, 2 * i + 1))],
        out_specs=pl.BlockSpec((_HB, W128), lambda i: (i, 0)),
        out_shape=jax.ShapeDtypeStruct((_SLOTS_ITEM, W128), jnp.float32),
    )(emb_t, emb_t)


_BNS = 16384


def _prep_bs(brand_t, shop_t):
    grid = (pl.cdiv(V_BS, _BNS),)
    return pl.pallas_call(
        _prep_pair_body,
        grid=grid,
        in_specs=[pl.BlockSpec((EMB, _BNS), lambda i: (0, i)),
                  pl.BlockSpec((EMB, _BNS), lambda i: (0, i))],
        out_specs=pl.BlockSpec((_BNS, W128), lambda i: (i, 0)),
        out_shape=jax.ShapeDtypeStruct((V_BS, W128), jnp.float32),
    )(brand_t, shop_t)


def _prep_pad_body(t_ref, out_ref):
    t = t_ref[...]
    out_ref[...] = jnp.concatenate(
        [t.T, jnp.zeros((t.shape[1], W128 - EMB), jnp.float32)], axis=1)


def _prep_cat(cat_t):
    return pl.pallas_call(
        _prep_pad_body,
        grid=(1,),
        in_specs=[pl.BlockSpec((EMB, V_CAT), lambda i: (0, 0))],
        out_specs=pl.BlockSpec((V_CAT, W128), lambda i: (0, 0)),
        out_shape=jax.ShapeDtypeStruct((V_CAT, W128), jnp.float32),
    )(cat_t)


def _mm_body(numt_ref, xi_ref, xc_ref, xbs_ref, ii_ref,
             wn_ref, wi_ref, wc_ref, wbs_ref, b_ref, out_ref):
    acc = lax.dot_general(numt_ref[...], wn_ref[...], (((0,), (0,)), ((), ())),
                          preferred_element_type=jnp.float32)
    h = ((ii_ref[...] >> 14) & 1).astype(jnp.float32)
    mlo = jnp.broadcast_to(1.0 - h, (h.shape[0], EMB))
    mhi = jnp.broadcast_to(h, (h.shape[0], EMB))
    msel = jnp.concatenate([mlo, mhi], axis=1)
    acc += jnp.dot(xi_ref[...] * msel, wi_ref[...],
                   preferred_element_type=jnp.float32)
    acc += jnp.dot(xc_ref[...], wc_ref[...], preferred_element_type=jnp.float32)
    acc += jnp.dot(xbs_ref[...], wbs_ref[...],
                   preferred_element_type=jnp.float32)
    out_ref[...] = acc + b_ref[...]


_BM = 2048


def _dense(num_t, xi, xc, xbs, ii, wn, wi, wc, wbs, b2):
    grid = (B // _BM,)
    row_spec = pl.BlockSpec((_BM, W128), lambda i: (i, 0))
    full = lambda s: pl.BlockSpec(s, lambda i: (0, 0))
    return pl.pallas_call(
        _mm_body,
        grid=grid,
        in_specs=[
            pl.BlockSpec((NUM, _BM), lambda i: (0, i)),
            row_spec, row_spec, row_spec,
            pl.BlockSpec((_BM, 1), lambda i: (i, 0)),
            full((NUM, HID)), full((W128, HID)), full((W128, HID)),
            full((W128, HID)), full((1, HID)),
        ],
        out_specs=pl.BlockSpec((_BM, HID), lambda i: (i, 0)),
        out_shape=jax.ShapeDtypeStruct((B, HID), jnp.float32),
    )(num_t, xi, xc, xbs, ii, wn, wi, wc, wbs, b2)


def kernel(numerical, idx_item_id, idx_category_id, idx_brand_id,
           idx_shop_id, emb_item_id, emb_category_id, emb_brand_id,
           emb_shop_id, W, b):
    vi = idx_item_id.astype(jnp.int32)
    slot_i = (vi >> 15) * _HB + (vi & (_HB - 1))
    to_idx = lambda i: i.astype(jnp.int32).reshape(-1, _CH)
    xi, xc, xbs = _gather(
        _prep_item(emb_item_id.T),
        _prep_cat(emb_category_id.T),
        _prep_bs(emb_brand_id.T, emb_shop_id.T),
        slot_i.reshape(-1, _CH), to_idx(idx_category_id),
        to_idx(idx_brand_id), to_idx(idx_shop_id))
    wt = W.T
    wi = jnp.concatenate([wt[NUM:NUM + EMB]] * 2, axis=0)
    wc = jnp.pad(wt[NUM + EMB:NUM + 2 * EMB], ((0, W128 - EMB), (0, 0)))
    wbs = wt[NUM + 2 * EMB:]
    return _dense(numerical.T, xi, xc, xbs, vi.reshape(B, 1),
                  wt[:NUM], wi, wc, wbs, b.reshape(1, HID))


# speedup vs baseline: 2.9346x; 1.0144x over previous
import functools

import jax
import jax.numpy as jnp
from jax import lax
from jax.experimental import pallas as pl
from jax.experimental.pallas import tpu as pltpu
from jax.experimental.pallas import tpu_sc as plsc

B = 16384
EMB = 64
NUM = 16
HID = 256
W128 = 128

_BN = 32768
_HB = _BN // 2
_CH = 128
_HC = 256

V_ITEM = 1000000
V_CAT = 1000
V_BS = 100000
_NBI = -(-V_ITEM // 32768)
_SLOTS_ITEM = _NBI * (32768 // 2)


def _build_gather():
    info = plsc.get_sparse_core_info()
    nc, ns = info.num_cores, info.num_subcores
    nw = nc * ns
    bw = B // nw
    n_ch = bw // _CH
    n_sub = bw // _HC
    n_hch = _HC // _CH

    mesh = plsc.VectorSubcoreMesh(core_axis_name="c", subcore_axis_name="s")

    scratch = [
        pltpu.VMEM((n_hch, _CH), jnp.int32),
        pltpu.VMEM((n_hch, _CH), jnp.int32),
        pltpu.VMEM((_HC, W128), jnp.float32),
        pltpu.VMEM((_HC, W128), jnp.float32),
        pltpu.SemaphoreType.DMA,
        pltpu.SemaphoreType.DMA,
    ]

    def make_fire(idx_ref):
        def fire(tbl, ih, crow, rows, sem, ibuf):
            pltpu.sync_copy(ih.at[pl.ds(crow, n_hch)], ibuf)
            copies = []
            for j in range(n_hch):
                copies.append(pltpu.async_copy(
                    tbl.at[ibuf.at[j]],
                    rows.at[pl.ds(j * _CH, _CH)], sem))
            return copies
        return fire

    @functools.partial(
        pl.kernel,
        mesh=mesh,
        compiler_params=pltpu.CompilerParams(use_tc_tiling_on_sc=False),
        out_type=[jax.ShapeDtypeStruct((B, W128), jnp.float32)] * 2,
        scratch_types=scratch,
    )
    def gather_small(t_cat, t_bs, i_cat, i_brand, i_shop,
                     o_cat, o_bs,
                     idx_h, idx_h2, rows_a, rows_b, sem_a, sem_b):
        wid = lax.axis_index("s") * nc + lax.axis_index("c")
        base = wid * bw
        fire = make_fire(None)

        for s in range(n_sub):
            crow = wid * n_ch + s * n_hch
            for c in fire(t_cat, i_cat, crow, rows_a, sem_a, idx_h):
                c.wait()
            pltpu.sync_copy(rows_a, o_cat.at[pl.ds(base + s * _HC, _HC)])

        def pack(i, _):
            for k in range(EMB // 16):
                rows_a[i, pl.ds(EMB + k * 16, 16)] = (
                    rows_b[i, pl.ds(EMB + k * 16, 16)])
            return 0

        for s in range(n_sub):
            crow = wid * n_ch + s * n_hch
            cps = fire(t_bs, i_brand, crow, rows_a, sem_a, idx_h)
            cps += fire(t_bs, i_shop, crow, rows_b, sem_b, idx_h2)
            for c in cps:
                c.wait()
            lax.fori_loop(0, _HC, pack, 0)
            pltpu.sync_copy(rows_a, o_bs.at[pl.ds(base + s * _HC, _HC)])

    @functools.partial(
        pl.kernel,
        mesh=mesh,
        compiler_params=pltpu.CompilerParams(use_tc_tiling_on_sc=False),
        out_type=jax.ShapeDtypeStruct((B, W128), jnp.float32),
        scratch_types=scratch,
    )
    def gather_item(t_item, i_item, o_item,
                    idx_h, idx_h2, rows_a, rows_b, sem_a, sem_b):
        wid = lax.axis_index("s") * nc + lax.axis_index("c")
        base = wid * bw
        fire = make_fire(None)
        for s in range(n_sub):
            crow = wid * n_ch + s * n_hch
            rows = rows_a if s % 2 == 0 else rows_b
            ib = idx_h if s % 2 == 0 else idx_h2
            sem = sem_a if s % 2 == 0 else sem_b
            for c in fire(t_item, i_item, crow, rows, sem, ib):
                c.wait()
            pltpu.sync_copy(rows, o_item.at[pl.ds(base + s * _HC, _HC)])

    return gather_small, gather_item


_gather_small, _gather_item = _build_gather()


def _prep_pair_body(t1_ref, t2_ref, out_ref):
    out_ref[...] = jnp.concatenate([t1_ref[...].T, t2_ref[...].T], axis=1)


def _prep_item(emb_t):
    grid = (_NBI,)
    return pl.pallas_call(
        _prep_pair_body,
        grid=grid,
        in_specs=[pl.BlockSpec((EMB, _HB), lambda i: (0, 2 * i)),
                  pl.BlockSpec((EMB, _HB), lambda i: (0, 2 * i + 1))],
        out_specs=pl.BlockSpec((_HB, W128), lambda i: (i, 0)),
        out_shape=jax.ShapeDtypeStruct((_SLOTS_ITEM, W128), jnp.float32),
    )(emb_t, emb_t)


_BNS = 16384


def _prep_bs(brand_t, shop_t):
    grid = (pl.cdiv(V_BS, _BNS),)
    return pl.pallas_call(
        _prep_pair_body,
        grid=grid,
        in_specs=[pl.BlockSpec((EMB, _BNS), lambda i: (0, i)),
                  pl.BlockSpec((EMB, _BNS), lambda i: (0, i))],
        out_specs=pl.BlockSpec((_BNS, W128), lambda i: (i, 0)),
        out_shape=jax.ShapeDtypeStruct((V_BS, W128), jnp.float32),
    )(brand_t, shop_t)


def _prep_pad_body(t_ref, out_ref):
    t = t_ref[...]
    out_ref[...] = jnp.concatenate(
        [t.T, jnp.zeros((t.shape[1], W128 - EMB), jnp.float32)], axis=1)


def _prep_cat(cat_t):
    return pl.pallas_call(
        _prep_pad_body,
        grid=(1,),
        in_specs=[pl.BlockSpec((EMB, V_CAT), lambda i: (0, 0))],
        out_specs=pl.BlockSpec((V_CAT, W128), lambda i: (0, 0)),
        out_shape=jax.ShapeDtypeStruct((V_CAT, W128), jnp.float32),
    )(cat_t)


def _mm_body(numt_ref, xi_ref, xc_ref, xbs_ref, ii_ref,
             wn_ref, wi_ref, wc_ref, wbs_ref, b_ref, out_ref):
    acc = lax.dot_general(numt_ref[...], wn_ref[...], (((0,), (0,)), ((), ())),
                          preferred_element_type=jnp.float32)
    h = ((ii_ref[...] >> 14) & 1).astype(jnp.float32)
    mlo = jnp.broadcast_to(1.0 - h, (h.shape[0], EMB))
    mhi = jnp.broadcast_to(h, (h.shape[0], EMB))
    msel = jnp.concatenate([mlo, mhi], axis=1)
    acc += jnp.dot(xi_ref[...] * msel, wi_ref[...],
                   preferred_element_type=jnp.float32)
    acc += jnp.dot(xc_ref[...], wc_ref[...], preferred_element_type=jnp.float32)
    acc += jnp.dot(xbs_ref[...], wbs_ref[...],
                   preferred_element_type=jnp.float32)
    out_ref[...] = acc + b_ref[...]


_BM = 2048


def _dense(num_t, xi, xc, xbs, ii, wn, wi, wc, wbs, b2):
    grid = (B // _BM,)
    row_spec = pl.BlockSpec((_BM, W128), lambda i: (i, 0))
    full = lambda s: pl.BlockSpec(s, lambda i: (0, 0))
    return pl.pallas_call(
        _mm_body,
        grid=grid,
        in_specs=[
            pl.BlockSpec((NUM, _BM), lambda i: (0, i)),
            row_spec, row_spec, row_spec,
            pl.BlockSpec((_BM, 1), lambda i: (i, 0)),
            full((NUM, HID)), full((W128, HID)), full((W128, HID)),
            full((W128, HID)), full((1, HID)),
        ],
        out_specs=pl.BlockSpec((_BM, HID), lambda i: (i, 0)),
        out_shape=jax.ShapeDtypeStruct((B, HID), jnp.float32),
    )(num_t, xi, xc, xbs, ii, wn, wi, wc, wbs, b2)


def kernel(numerical, idx_item_id, idx_category_id, idx_brand_id,
           idx_shop_id, emb_item_id, emb_category_id, emb_brand_id,
           emb_shop_id, W, b):
    vi = idx_item_id.astype(jnp.int32)
    slot_i = (vi >> 15) * _HB + (vi & (_HB - 1))
    to_idx = lambda i: i.astype(jnp.int32).reshape(-1, _CH)
    xc, xbs = _gather_small(
        _prep_cat(emb_category_id.T),
        _prep_bs(emb_brand_id.T, emb_shop_id.T),
        to_idx(idx_category_id), to_idx(idx_brand_id),
        to_idx(idx_shop_id))
    xi = _gather_item(_prep_item(emb_item_id.T), slot_i.reshape(-1, _CH))
    wt = W.T
    wi = jnp.concatenate([wt[NUM:NUM + EMB]] * 2, axis=0)
    wc = jnp.pad(wt[NUM + EMB:NUM + 2 * EMB], ((0, W128 - EMB), (0, 0)))
    wbs = wt[NUM + 2 * EMB:]
    return _dense(numerical.T, xi, xc, xbs, vi.reshape(B, 1),
                  wt[:NUM], wi, wc, wbs, b.reshape(1, HID))


# speedup vs baseline: 3.5709x; 1.2168x over previous
import functools

import jax
import jax.numpy as jnp
from jax import lax
from jax.experimental import pallas as pl
from jax.experimental.pallas import tpu as pltpu
from jax.experimental.pallas import tpu_sc as plsc

B = 16384
EMB = 64
NUM = 16
HID = 256
W128 = 128

_BN = 32768
_HB = _BN // 2
_CH = 128
_HC = 256

V_ITEM = 1000000
V_CAT = 1000
V_BS = 100000
_NBI = -(-V_ITEM // 32768)
_SLOTS_ITEM = _NBI * (32768 // 2)


def _build_gather():
    info = plsc.get_sparse_core_info()
    nc, ns = info.num_cores, info.num_subcores
    nw = nc * ns
    bw = B // nw
    n_ch = bw // _CH
    n_sub = bw // _HC
    n_hch = _HC // _CH

    mesh = plsc.VectorSubcoreMesh(core_axis_name="c", subcore_axis_name="s")

    scratch = [
        pltpu.VMEM((n_hch, _CH), jnp.int32),
        pltpu.VMEM((n_hch, _CH), jnp.int32),
        pltpu.VMEM((_HC, W128), jnp.float32),
        pltpu.VMEM((_HC, W128), jnp.float32),
        pltpu.SemaphoreType.DMA,
        pltpu.SemaphoreType.DMA,
    ]

    def make_fire(idx_ref):
        def fire(tbl, ih, crow, rows, sem, ibuf):
            pltpu.sync_copy(ih.at[pl.ds(crow, n_hch)], ibuf)
            copies = []
            for j in range(n_hch):
                copies.append(pltpu.async_copy(
                    tbl.at[ibuf.at[j]],
                    rows.at[pl.ds(j * _CH, _CH)], sem))
            return copies
        return fire

    @functools.partial(
        pl.kernel,
        mesh=mesh,
        compiler_params=pltpu.CompilerParams(use_tc_tiling_on_sc=False),
        out_type=[jax.ShapeDtypeStruct((B, W128), jnp.float32)] * 2,
        scratch_types=scratch,
    )
    def gather_small(t_cat, t_bs, i_cat, i_brand, i_shop,
                     o_cat, o_bs,
                     idx_h, idx_h2, rows_a, rows_b, sem_a, sem_b):
        wid = lax.axis_index("s") * nc + lax.axis_index("c")
        base = wid * bw
        fire = make_fire(None)

        for s in range(n_sub):
            crow = wid * n_ch + s * n_hch
            for c in fire(t_cat, i_cat, crow, rows_a, sem_a, idx_h):
                c.wait()
            pltpu.sync_copy(rows_a, o_cat.at[pl.ds(base + s * _HC, _HC)])

        def pack(i, _):
            for k in range(EMB // 16):
                rows_a[i, pl.ds(EMB + k * 16, 16)] = (
                    rows_b[i, pl.ds(EMB + k * 16, 16)])
            return 0

        for s in range(n_sub):
            crow = wid * n_ch + s * n_hch
            cps = fire(t_bs, i_brand, crow, rows_a, sem_a, idx_h)
            cps += fire(t_bs, i_shop, crow, rows_b, sem_b, idx_h2)
            for c in cps:
                c.wait()
            lax.fori_loop(0, _HC, pack, 0)
            pltpu.sync_copy(rows_a, o_bs.at[pl.ds(base + s * _HC, _HC)])

    @functools.partial(
        pl.kernel,
        mesh=mesh,
        compiler_params=pltpu.CompilerParams(use_tc_tiling_on_sc=False),
        out_type=jax.ShapeDtypeStruct((B, W128), jnp.float32),
        scratch_types=scratch,
    )
    def gather_item(t_item, i_item, o_item,
                    idx_h, idx_h2, rows_a, rows_b, sem_a, sem_b):
        wid = lax.axis_index("s") * nc + lax.axis_index("c")
        base = wid * bw
        fire = make_fire(None)
        for s in range(n_sub):
            crow = wid * n_ch + s * n_hch
            rows = rows_a if s % 2 == 0 else rows_b
            ib = idx_h if s % 2 == 0 else idx_h2
            sem = sem_a if s % 2 == 0 else sem_b
            for c in fire(t_item, i_item, crow, rows, sem, ib):
                c.wait()
            pltpu.sync_copy(rows, o_item.at[pl.ds(base + s * _HC, _HC)])

    return gather_small, gather_item


_gather_small, _gather_item = _build_gather()


def _prep_pair_body(t1_ref, t2_ref, out_ref):
    out_ref[:, :EMB] = t1_ref[...].astype(jnp.bfloat16).T.astype(jnp.float32)
    out_ref[:, EMB:] = t2_ref[...].astype(jnp.bfloat16).T.astype(jnp.float32)


def _prep_item(emb_t):
    grid = (_NBI,)
    return pl.pallas_call(
        _prep_pair_body,
        grid=grid,
        in_specs=[pl.BlockSpec((EMB, _HB), lambda i: (0, 2 * i)),
                  pl.BlockSpec((EMB, _HB), lambda i: (0, 2 * i + 1))],
        out_specs=pl.BlockSpec((_HB, W128), lambda i: (i, 0)),
        out_shape=jax.ShapeDtypeStruct((_SLOTS_ITEM, W128), jnp.float32),
    )(emb_t, emb_t)


_BNS = 16384


def _prep_bs(brand_t, shop_t):
    grid = (pl.cdiv(V_BS, _BNS),)
    return pl.pallas_call(
        _prep_pair_body,
        grid=grid,
        in_specs=[pl.BlockSpec((EMB, _BNS), lambda i: (0, i)),
                  pl.BlockSpec((EMB, _BNS), lambda i: (0, i))],
        out_specs=pl.BlockSpec((_BNS, W128), lambda i: (i, 0)),
        out_shape=jax.ShapeDtypeStruct((V_BS, W128), jnp.float32),
    )(brand_t, shop_t)


def _prep_pad_body(t_ref, out_ref):
    t = t_ref[...]
    out_ref[...] = jnp.concatenate(
        [t.T, jnp.zeros((t.shape[1], W128 - EMB), jnp.float32)], axis=1)


def _prep_cat(cat_t):
    return pl.pallas_call(
        _prep_pad_body,
        grid=(1,),
        in_specs=[pl.BlockSpec((EMB, V_CAT), lambda i: (0, 0))],
        out_specs=pl.BlockSpec((V_CAT, W128), lambda i: (0, 0)),
        out_shape=jax.ShapeDtypeStruct((V_CAT, W128), jnp.float32),
    )(cat_t)


def _mm_body(numt_ref, xi_ref, xc_ref, xbs_ref, ii_ref,
             wn_ref, wi_ref, wc_ref, wbs_ref, b_ref, out_ref):
    acc = lax.dot_general(numt_ref[...], wn_ref[...], (((0,), (0,)), ((), ())),
                          preferred_element_type=jnp.float32)
    h = ((ii_ref[...] >> 14) & 1).astype(jnp.float32)
    mlo = jnp.broadcast_to(1.0 - h, (h.shape[0], EMB))
    mhi = jnp.broadcast_to(h, (h.shape[0], EMB))
    msel = jnp.concatenate([mlo, mhi], axis=1)
    acc += jnp.dot(xi_ref[...] * msel, wi_ref[...],
                   preferred_element_type=jnp.float32)
    acc += jnp.dot(xc_ref[...], wc_ref[...], preferred_element_type=jnp.float32)
    acc += jnp.dot(xbs_ref[...], wbs_ref[...],
                   preferred_element_type=jnp.float32)
    out_ref[...] = acc + b_ref[...]


_BM = 2048


def _dense(num_t, xi, xc, xbs, ii, wn, wi, wc, wbs, b2):
    grid = (B // _BM,)
    row_spec = pl.BlockSpec((_BM, W128), lambda i: (i, 0))
    full = lambda s: pl.BlockSpec(s, lambda i: (0, 0))
    return pl.pallas_call(
        _mm_body,
        grid=grid,
        in_specs=[
            pl.BlockSpec((NUM, _BM), lambda i: (0, i)),
            row_spec, row_spec, row_spec,
            pl.BlockSpec((_BM, 1), lambda i: (i, 0)),
            full((NUM, HID)), full((W128, HID)), full((W128, HID)),
            full((W128, HID)), full((1, HID)),
        ],
        out_specs=pl.BlockSpec((_BM, HID), lambda i: (i, 0)),
        out_shape=jax.ShapeDtypeStruct((B, HID), jnp.float32),
    )(num_t, xi, xc, xbs, ii, wn, wi, wc, wbs, b2)


def kernel(numerical, idx_item_id, idx_category_id, idx_brand_id,
           idx_shop_id, emb_item_id, emb_category_id, emb_brand_id,
           emb_shop_id, W, b):
    vi = idx_item_id.astype(jnp.int32)
    slot_i = (vi >> 15) * _HB + (vi & (_HB - 1))
    to_idx = lambda i: i.astype(jnp.int32).reshape(-1, _CH)
    xc, xbs = _gather_small(
        _prep_cat(emb_category_id.T),
        _prep_bs(emb_brand_id.T, emb_shop_id.T),
        to_idx(idx_category_id), to_idx(idx_brand_id),
        to_idx(idx_shop_id))
    xi = _gather_item(_prep_item(emb_item_id.T), slot_i.reshape(-1, _CH))
    wt = W.T
    wi = jnp.concatenate([wt[NUM:NUM + EMB]] * 2, axis=0)
    wc = jnp.pad(wt[NUM + EMB:NUM + 2 * EMB], ((0, W128 - EMB), (0, 0)))
    wbs = wt[NUM + 2 * EMB:]
    return _dense(numerical.T, xi, xc, xbs, vi.reshape(B, 1),
                  wt[:NUM], wi, wc, wbs, b.reshape(1, HID))


# speedup vs baseline: 3.5719x; 1.0003x over previous
import functools

import jax
import jax.numpy as jnp
from jax import lax
from jax.experimental import pallas as pl
from jax.experimental.pallas import tpu as pltpu
from jax.experimental.pallas import tpu_sc as plsc

B = 16384
EMB = 64
NUM = 16
HID = 256
W128 = 128

_BN = 32768
_HB = _BN // 2
_CH = 128
_HC = 256

V_ITEM = 1000000
V_CAT = 1000
V_BS = 100000
_NBI = -(-V_ITEM // 32768)
_SLOTS_ITEM = _NBI * (32768 // 2)


def _build_gather():
    info = plsc.get_sparse_core_info()
    nc, ns = info.num_cores, info.num_subcores
    nw = nc * ns
    bw = B // nw
    n_ch = bw // _CH
    n_sub = bw // _HC
    n_hch = _HC // _CH

    mesh = plsc.VectorSubcoreMesh(core_axis_name="c", subcore_axis_name="s")

    scratch = [
        pltpu.VMEM((n_hch, _CH), jnp.int32),
        pltpu.VMEM((n_hch, _CH), jnp.int32),
        pltpu.VMEM((_HC, W128), jnp.float32),
        pltpu.VMEM((_HC, W128), jnp.float32),
        pltpu.SemaphoreType.DMA,
        pltpu.SemaphoreType.DMA,
    ]

    def fire(tbl, ih, crow, rows, sem, ibuf):
        pltpu.sync_copy(ih.at[pl.ds(crow, n_hch)], ibuf)
        copies = []
        for j in range(n_hch):
            copies.append(pltpu.async_copy(
                tbl.at[ibuf.at[j]],
                rows.at[pl.ds(j * _CH, _CH)], sem))
        return copies

    @functools.partial(
        pl.kernel,
        mesh=mesh,
        compiler_params=pltpu.CompilerParams(use_tc_tiling_on_sc=False),
        out_type=[jax.ShapeDtypeStruct((B, W128), jnp.float32)] * 2,
        scratch_types=scratch,
    )
    def gather_small(t_cat, t_bs, i_cat, i_brand, i_shop,
                     o_cat, o_bs,
                     idx_h, idx_h2, rows_a, rows_b, sem_a, sem_b):
        wid = lax.axis_index("s") * nc + lax.axis_index("c")
        base = wid * bw

        for s in range(n_sub):
            crow = wid * n_ch + s * n_hch
            for c in fire(t_cat, i_cat, crow, rows_a, sem_a, idx_h):
                c.wait()
            pltpu.sync_copy(rows_a, o_cat.at[pl.ds(base + s * _HC, _HC)])

        def pack(i, _):
            for k in range(EMB // 16):
                rows_a[i, pl.ds(EMB + k * 16, 16)] = (
                    rows_b[i, pl.ds(EMB + k * 16, 16)])
            return 0

        for s in range(n_sub):
            crow = wid * n_ch + s * n_hch
            cps = fire(t_bs, i_brand, crow, rows_a, sem_a, idx_h)
            cps += fire(t_bs, i_shop, crow, rows_b, sem_b, idx_h2)
            for c in cps:
                c.wait()
            lax.fori_loop(0, _HC, pack, 0)
            pltpu.sync_copy(rows_a, o_bs.at[pl.ds(base + s * _HC, _HC)])

    @functools.partial(
        pl.kernel,
        mesh=mesh,
        compiler_params=pltpu.CompilerParams(use_tc_tiling_on_sc=False),
        out_type=jax.ShapeDtypeStruct((B, W128), jnp.float32),
        scratch_types=scratch,
    )
    def gather_item(t_item, i_item, o_item,
                    idx_h, idx_h2, rows_a, rows_b, sem_a, sem_b):
        wid = lax.axis_index("s") * nc + lax.axis_index("c")
        base = wid * bw
        for s in range(n_sub):
            crow = wid * n_ch + s * n_hch
            rows = rows_a if s % 2 == 0 else rows_b
            ib = idx_h if s % 2 == 0 else idx_h2
            sem = sem_a if s % 2 == 0 else sem_b
            for c in fire(t_item, i_item, crow, rows, sem, ib):
                c.wait()
            pltpu.sync_copy(rows, o_item.at[pl.ds(base + s * _HC, _HC)])

    return gather_small, gather_item


_gather_small, _gather_item = _build_gather()


def _prep_pair_body(t1_ref, t2_ref, out_ref):
    out_ref[:, :EMB] = t1_ref[...].astype(jnp.bfloat16).T.astype(jnp.float32)
    out_ref[:, EMB:] = t2_ref[...].astype(jnp.bfloat16).T.astype(jnp.float32)


def _prep_item(emb_t):
    grid = (_NBI,)
    return pl.pallas_call(
        _prep_pair_body,
        grid=grid,
        in_specs=[pl.BlockSpec((EMB, _HB), lambda i: (0, 2 * i)),
                  pl.BlockSpec((EMB, _HB), lambda i: (0, 2 * i + 1))],
        out_specs=pl.BlockSpec((_HB, W128), lambda i: (i, 0)),
        out_shape=jax.ShapeDtypeStruct((_SLOTS_ITEM, W128), jnp.float32),
    )(emb_t, emb_t)


_BNS = 16384


def _prep_bs(brand_t, shop_t):
    grid = (pl.cdiv(V_BS, _BNS),)
    return pl.pallas_call(
        _prep_pair_body,
        grid=grid,
        in_specs=[pl.BlockSpec((EMB, _BNS), lambda i: (0, i)),
                  pl.BlockSpec((EMB, _BNS), lambda i: (0, i))],
        out_specs=pl.BlockSpec((_BNS, W128), lambda i: (i, 0)),
        out_shape=jax.ShapeDtypeStruct((V_BS, W128), jnp.float32),
    )(brand_t, shop_t)


def _prep_pad_body(t_ref, out_ref):
    t = t_ref[...]
    out_ref[...] = jnp.concatenate(
        [t.T, jnp.zeros((t.shape[1], W128 - EMB), jnp.float32)], axis=1)


def _prep_cat(cat_t):
    return pl.pallas_call(
        _prep_pad_body,
        grid=(1,),
        in_specs=[pl.BlockSpec((EMB, V_CAT), lambda i: (0, 0))],
        out_specs=pl.BlockSpec((V_CAT, W128), lambda i: (0, 0)),
        out_shape=jax.ShapeDtypeStruct((V_CAT, W128), jnp.float32),
    )(cat_t)


def _mm_body(numt_ref, xi_ref, xc_ref, xbs_ref, ii_ref,
             wn_ref, wi_ref, wc_ref, wbs_ref, b_ref, out_ref):
    acc = lax.dot_general(numt_ref[...], wn_ref[...], (((0,), (0,)), ((), ())),
                          preferred_element_type=jnp.float32)
    h = ((ii_ref[...] >> 14) & 1).astype(jnp.float32)
    mlo = jnp.broadcast_to(1.0 - h, (h.shape[0], EMB))
    mhi = jnp.broadcast_to(h, (h.shape[0], EMB))
    msel = jnp.concatenate([mlo, mhi], axis=1)
    acc += jnp.dot(xi_ref[...] * msel, wi_ref[...],
                   preferred_element_type=jnp.float32)
    acc += jnp.dot(xc_ref[...], wc_ref[...], preferred_element_type=jnp.float32)
    acc += jnp.dot(xbs_ref[...], wbs_ref[...],
                   preferred_element_type=jnp.float32)
    out_ref[...] = acc + b_ref[...]


_BM = 2048


def _dense(num_t, xi, xc, xbs, ii, wn, wi, wc, wbs, b2):
    grid = (B // _BM,)
    row_spec = pl.BlockSpec((_BM, W128), lambda i: (i, 0))
    full = lambda s: pl.BlockSpec(s, lambda i: (0, 0))
    return pl.pallas_call(
        _mm_body,
        grid=grid,
        in_specs=[
            pl.BlockSpec((NUM, _BM), lambda i: (0, i)),
            row_spec, row_spec, row_spec,
            pl.BlockSpec((_BM, 1), lambda i: (i, 0)),
            full((NUM, HID)), full((W128, HID)), full((W128, HID)),
            full((W128, HID)), full((1, HID)),
        ],
        out_specs=pl.BlockSpec((_BM, HID), lambda i: (i, 0)),
        out_shape=jax.ShapeDtypeStruct((B, HID), jnp.float32),
    )(num_t, xi, xc, xbs, ii, wn, wi, wc, wbs, b2)


def kernel(numerical, idx_item_id, idx_category_id, idx_brand_id,
           idx_shop_id, emb_item_id, emb_category_id, emb_brand_id,
           emb_shop_id, W, b):
    vi = idx_item_id.astype(jnp.int32)
    slot_i = (vi >> 15) * _HB + (vi & (_HB - 1))
    to_idx = lambda i: i.astype(jnp.int32).reshape(-1, _CH)
    xc, xbs = _gather_small(
        _prep_cat(emb_category_id.T),
        _prep_bs(emb_brand_id.T, emb_shop_id.T),
        to_idx(idx_category_id), to_idx(idx_brand_id),
        to_idx(idx_shop_id))
    xi = _gather_item(_prep_item(emb_item_id.T), slot_i.reshape(-1, _CH))
    wt = W.T
    wi = jnp.concatenate([wt[NUM:NUM + EMB]] * 2, axis=0)
    wc = jnp.pad(wt[NUM + EMB:NUM + 2 * EMB], ((0, W128 - EMB), (0, 0)))
    wbs = wt[NUM + 2 * EMB:]
    return _dense(numerical.T, xi, xc, xbs, vi.reshape(B, 1),
                  wt[:NUM], wi, wc, wbs, b.reshape(1, HID))
